# baseline (speedup 1.0000x reference)
import functools

import jax
import jax.numpy as jnp
from jax import lax
from jax.experimental import pallas as pl
from jax.experimental.pallas import tpu as pltpu
from jax.experimental.pallas import tpu_sc as plsc

N = 10000
E = 320000
D = 128
NC = 2
NT = 16
K = 80
C_EDGE = E // (NC * NT * K)
NB = 2
CB = 25
RPT = 640
RPT_LAST = N - (NT - 1) * RPT
DEG_PAD = 10240
DEG_TILE = DEG_PAD // NT


def _sc_mesh():
    return plsc.VectorSubcoreMesh(core_axis_name="c", subcore_axis_name="s")


@functools.cache
def _build_deg_kernel():
    return functools.partial(
        pl.kernel,
        out_type=jax.ShapeDtypeStruct((NC, DEG_PAD), jnp.float32),
        mesh=_sc_mesh(),
        scratch_types=[
            pltpu.VMEM((CB, K), jnp.int32),
            pltpu.VMEM((K,), jnp.float32),
            pltpu.VMEM((DEG_TILE,), jnp.float32),
            pltpu.VMEM_SHARED((DEG_PAD,), jnp.float32),
        ],
    )(_deg_body)


def _deg_body(dst_hbm, out_hbm, idx_v, ones_v, zer_v, dacc):
    c = lax.axis_index("c")
    s = lax.axis_index("s")
    for i in range(K // 16):
        ones_v[pl.ds(i * 16, 16)] = jnp.ones((16,), jnp.float32)
    for i in range(DEG_TILE // 16):
        zer_v[pl.ds(i * 16, 16)] = jnp.zeros((16,), jnp.float32)
    d0 = pl.multiple_of(s * DEG_TILE, 8)
    pltpu.sync_copy(zer_v, dacc.at[pl.ds(d0, DEG_TILE)])
    plsc.subcore_barrier()

    def block(blk, carry):
        pltpu.sync_copy(dst_hbm.at[c, s, blk], idx_v)

        def body(j, carry2):
            pltpu.sync_copy(ones_v, dacc.at[idx_v.at[j]], add=True)
            return carry2

        lax.fori_loop(0, CB, body, 0)
        return carry

    lax.fori_loop(0, C_EDGE // CB, block, 0)
    plsc.subcore_barrier()
    pltpu.sync_copy(dacc.at[pl.ds(d0, DEG_TILE)],
                    out_hbm.at[c, pl.ds(d0, DEG_TILE)])


@functools.cache
def _build_agg_kernel():
    return functools.partial(
        pl.kernel,
        out_type=jax.ShapeDtypeStruct((NC, N, D), jnp.float32),
        mesh=_sc_mesh(),
        scratch_types=[
            pltpu.VMEM((CB, K), jnp.int32),
            pltpu.VMEM((CB, K), jnp.int32),
            *[pltpu.VMEM((K, D), jnp.float32) for _ in range(NB)],
            pltpu.VMEM_SHARED((N, D), jnp.float32),
            pltpu.SemaphoreType.DMA,
        ],
    )(_agg_body)


def _agg_body(hp_hbm, src_hbm, dst_hbm, out_hbm,
              src_v, dst_v, *rest):
    bufs = rest[:NB]
    acc, gsem = rest[NB], rest[NB + 1]
    c = lax.axis_index("c")
    s = lax.axis_index("s")
    r0 = pl.multiple_of(s * RPT, 8)

    @pl.when(s < NT - 1)
    def _stage_full():
        pltpu.sync_copy(hp_hbm.at[pl.ds(r0, RPT)], acc.at[pl.ds(r0, RPT)])

    @pl.when(s == NT - 1)
    def _stage_last():
        r = (NT - 1) * RPT
        pltpu.sync_copy(hp_hbm.at[pl.ds(r, RPT_LAST)],
                        acc.at[pl.ds(r, RPT_LAST)])

    plsc.subcore_barrier()

    def _wait_gather(j, buf):
        pltpu.make_async_copy(hp_hbm.at[src_v.at[j]], buf, gsem).wait()

    def block(blk, carry):
        pltpu.sync_copy(src_hbm.at[c, s, blk], src_v)
        pltpu.sync_copy(dst_hbm.at[c, s, blk], dst_v)
        pltpu.async_copy(hp_hbm.at[src_v.at[0]], bufs[0], gsem)

        def body(i, carry2):
            j = 2 * i
            _wait_gather(j, bufs[0])
            pltpu.async_copy(hp_hbm.at[src_v.at[j + 1]], bufs[1], gsem)
            pltpu.sync_copy(bufs[0], acc.at[dst_v.at[j]], add=True)
            _wait_gather(j + 1, bufs[1])
            pltpu.async_copy(hp_hbm.at[src_v.at[j + 2]], bufs[0], gsem)
            pltpu.sync_copy(bufs[1], acc.at[dst_v.at[j + 1]], add=True)
            return carry2

        lax.fori_loop(0, (CB - 1) // 2, body, 0)
        _wait_gather(CB - 1, bufs[0])
        pltpu.sync_copy(bufs[0], acc.at[dst_v.at[CB - 1]], add=True)
        return carry

    lax.fori_loop(0, C_EDGE // CB, block, 0)
    plsc.subcore_barrier()

    @pl.when(s < NT - 1)
    def _write_full():
        pltpu.sync_copy(acc.at[pl.ds(r0, RPT)], out_hbm.at[c, pl.ds(r0, RPT)])

    @pl.when(s == NT - 1)
    def _write_last():
        r = (NT - 1) * RPT
        pltpu.sync_copy(acc.at[pl.ds(r, RPT_LAST)],
                        out_hbm.at[c, pl.ds(r, RPT_LAST)])


_BR = 2000


def _mm1_body(x_ref, w_ref, u_ref):
    u_ref[...] = jnp.dot(x_ref[...], w_ref[...],
                         preferred_element_type=jnp.float32)


def _pre_body(dp_ref, u_ref, dis_ref, hp_ref):
    deg = dp_ref[0] + dp_ref[1] + 1.0
    dis = lax.rsqrt(deg)
    dis_ref[...] = dis
    hp_ref[...] = u_ref[...] * dis


def _mid_body(s_ref, hp_ref, dis_ref, b_ref, w_ref, o_ref):
    dis = dis_ref[...]
    agg = s_ref[0] + s_ref[1] - hp_ref[...]
    t = jnp.maximum(dis * agg + b_ref[...], 0.0)
    u = jnp.dot(t, w_ref[...], preferred_element_type=jnp.float32)
    o_ref[...] = u * dis


def _post_body(s_ref, hp_ref, dis_ref, b_ref, o_ref):
    agg = s_ref[0] + s_ref[1] - hp_ref[...]
    o_ref[...] = dis_ref[...] * agg + b_ref[...]


def _mm1_call(x, W1):
    return pl.pallas_call(
        _mm1_body,
        grid=(N // _BR,),
        in_specs=[
            pl.BlockSpec((_BR, D), lambda i: (i, 0)),
            pl.BlockSpec((D, D), lambda i: (0, 0)),
        ],
        out_specs=pl.BlockSpec((_BR, D), lambda i: (i, 0)),
        out_shape=jax.ShapeDtypeStruct((N, D), jnp.float32),
    )(x, W1)


def _pre_call(degp, u1):
    return pl.pallas_call(
        _pre_body,
        grid=(N // _BR,),
        in_specs=[
            pl.BlockSpec((NC, _BR, 1), lambda i: (0, i, 0)),
            pl.BlockSpec((_BR, D), lambda i: (i, 0)),
        ],
        out_specs=[
            pl.BlockSpec((_BR, 1), lambda i: (i, 0)),
            pl.BlockSpec((_BR, D), lambda i: (i, 0)),
        ],
        out_shape=[
            jax.ShapeDtypeStruct((N, 1), jnp.float32),
            jax.ShapeDtypeStruct((N, D), jnp.float32),
        ],
    )(degp, u1)


def _mid_call(s1, hp1, dis, b1, W2):
    return pl.pallas_call(
        _mid_body,
        grid=(N // _BR,),
        in_specs=[
            pl.BlockSpec((NC, _BR, D), lambda i: (0, i, 0)),
            pl.BlockSpec((_BR, D), lambda i: (i, 0)),
            pl.BlockSpec((_BR, 1), lambda i: (i, 0)),
            pl.BlockSpec((1, D), lambda i: (0, 0)),
            pl.BlockSpec((D, D), lambda i: (0, 0)),
        ],
        out_specs=pl.BlockSpec((_BR, D), lambda i: (i, 0)),
        out_shape=jax.ShapeDtypeStruct((N, D), jnp.float32),
    )(s1, hp1, dis, b1, W2)


def _post_call(s2, hp2, dis, b2):
    return pl.pallas_call(
        _post_body,
        grid=(N // _BR,),
        in_specs=[
            pl.BlockSpec((NC, _BR, D), lambda i: (0, i, 0)),
            pl.BlockSpec((_BR, D), lambda i: (i, 0)),
            pl.BlockSpec((_BR, 1), lambda i: (i, 0)),
            pl.BlockSpec((1, D), lambda i: (0, 0)),
        ],
        out_specs=pl.BlockSpec((_BR, D), lambda i: (i, 0)),
        out_shape=jax.ShapeDtypeStruct((N, D), jnp.float32),
    )(s2, hp2, dis, b2)


def kernel(x, edge_index, W1, b1, W2, b2):
    src = edge_index[0].astype(jnp.int32)
    dst = edge_index[1].astype(jnp.int32)
    src4 = src.reshape(NC, NT, C_EDGE // CB, CB, K)
    dst4 = dst.reshape(NC, NT, C_EDGE // CB, CB, K)

    degp = _build_deg_kernel()(dst4)
    u1 = _mm1_call(x, W1)
    degp = degp[:, :N, None]
    dis, hp1 = _pre_call(degp, u1)
    s1 = _build_agg_kernel()(hp1, src4, dst4)
    hp2 = _mid_call(s1, hp1, dis, b1.reshape(1, D), W2)
    s2 = _build_agg_kernel()(hp2, src4, dst4)
    return _post_call(s2, hp2, dis, b2.reshape(1, D))


# speedup vs baseline: 1.0103x; 1.0103x over previous
import functools

import jax
import jax.numpy as jnp
from jax import lax
from jax.experimental import pallas as pl
from jax.experimental.pallas import tpu as pltpu
from jax.experimental.pallas import tpu_sc as plsc

N = 10000
E = 320000
D = 128
NC = 2
NT = 16
K = 80
C_EDGE = E // (NC * NT * K)
NB = 2
CB = 25
RPT = 640
RPT_LAST = N - (NT - 1) * RPT
DEG_PAD = 10240
DEG_TILE = DEG_PAD // NT


def _sc_mesh():
    return plsc.VectorSubcoreMesh(core_axis_name="c", subcore_axis_name="s")


@functools.cache
def _build_deg_kernel():
    return functools.partial(
        pl.kernel,
        out_type=jax.ShapeDtypeStruct((NC, DEG_PAD), jnp.float32),
        mesh=_sc_mesh(),
        scratch_types=[
            pltpu.VMEM((CB, K), jnp.int32),
            pltpu.VMEM((K,), jnp.float32),
            pltpu.VMEM((DEG_TILE,), jnp.float32),
            pltpu.VMEM_SHARED((DEG_PAD,), jnp.float32),
            pltpu.SemaphoreType.DMA,
        ],
    )(_deg_body)


def _deg_body(dst_hbm, out_hbm, idx_v, ones_v, zer_v, dacc, dsem):
    c = lax.axis_index("c")
    s = lax.axis_index("s")
    for i in range(K // 16):
        ones_v[pl.ds(i * 16, 16)] = jnp.ones((16,), jnp.float32)
    for i in range(DEG_TILE // 16):
        zer_v[pl.ds(i * 16, 16)] = jnp.zeros((16,), jnp.float32)
    d0 = pl.multiple_of(s * DEG_TILE, 8)
    pltpu.sync_copy(zer_v, dacc.at[pl.ds(d0, DEG_TILE)])
    plsc.subcore_barrier()

    def block(blk, carry):
        pltpu.sync_copy(dst_hbm.at[c, s, blk], idx_v)
        pltpu.async_copy(ones_v, dacc.at[idx_v.at[0]], dsem, add=True)

        def body(j, carry2):
            pltpu.async_copy(ones_v, dacc.at[idx_v.at[j]], dsem, add=True)
            pltpu.make_async_copy(ones_v, dacc.at[idx_v.at[j - 1]],
                                  dsem).wait()
            return carry2

        lax.fori_loop(1, CB, body, 0)
        pltpu.make_async_copy(ones_v, dacc.at[idx_v.at[CB - 1]], dsem).wait()
        return carry

    lax.fori_loop(0, C_EDGE // CB, block, 0)
    plsc.subcore_barrier()
    pltpu.sync_copy(dacc.at[pl.ds(d0, DEG_TILE)],
                    out_hbm.at[c, pl.ds(d0, DEG_TILE)])


@functools.cache
def _build_agg_kernel():
    return functools.partial(
        pl.kernel,
        out_type=jax.ShapeDtypeStruct((NC, N, D), jnp.float32),
        mesh=_sc_mesh(),
        scratch_types=[
            pltpu.VMEM((CB, K), jnp.int32),
            pltpu.VMEM((CB, K), jnp.int32),
            *[pltpu.VMEM((K, D), jnp.float32) for _ in range(NB)],
            pltpu.VMEM_SHARED((N, D), jnp.float32),
            pltpu.SemaphoreType.DMA,
        ],
    )(_agg_body)


def _agg_body(hp_hbm, src_hbm, dst_hbm, out_hbm,
              src_v, dst_v, *rest):
    bufs = rest[:NB]
    acc, gsem = rest[NB], rest[NB + 1]
    c = lax.axis_index("c")
    s = lax.axis_index("s")
    r0 = pl.multiple_of(s * RPT, 8)

    @pl.when(s < NT - 1)
    def _stage_full():
        pltpu.sync_copy(hp_hbm.at[pl.ds(r0, RPT)], acc.at[pl.ds(r0, RPT)])

    @pl.when(s == NT - 1)
    def _stage_last():
        r = (NT - 1) * RPT
        pltpu.sync_copy(hp_hbm.at[pl.ds(r, RPT_LAST)],
                        acc.at[pl.ds(r, RPT_LAST)])

    plsc.subcore_barrier()

    def _wait_gather(j, buf):
        pltpu.make_async_copy(hp_hbm.at[src_v.at[j]], buf, gsem).wait()

    def block(blk, carry):
        pltpu.sync_copy(src_hbm.at[c, s, blk], src_v)
        pltpu.sync_copy(dst_hbm.at[c, s, blk], dst_v)
        pltpu.async_copy(hp_hbm.at[src_v.at[0]], bufs[0], gsem)

        def body(i, carry2):
            j = 2 * i
            _wait_gather(j, bufs[0])
            pltpu.async_copy(hp_hbm.at[src_v.at[j + 1]], bufs[1], gsem)
            pltpu.sync_copy(bufs[0], acc.at[dst_v.at[j]], add=True)
            _wait_gather(j + 1, bufs[1])
            pltpu.async_copy(hp_hbm.at[src_v.at[j + 2]], bufs[0], gsem)
            pltpu.sync_copy(bufs[1], acc.at[dst_v.at[j + 1]], add=True)
            return carry2

        lax.fori_loop(0, (CB - 1) // 2, body, 0)
        _wait_gather(CB - 1, bufs[0])
        pltpu.sync_copy(bufs[0], acc.at[dst_v.at[CB - 1]], add=True)
        return carry

    lax.fori_loop(0, C_EDGE // CB, block, 0)
    plsc.subcore_barrier()

    @pl.when(s < NT - 1)
    def _write_full():
        pltpu.sync_copy(acc.at[pl.ds(r0, RPT)], out_hbm.at[c, pl.ds(r0, RPT)])

    @pl.when(s == NT - 1)
    def _write_last():
        r = (NT - 1) * RPT
        pltpu.sync_copy(acc.at[pl.ds(r, RPT_LAST)],
                        out_hbm.at[c, pl.ds(r, RPT_LAST)])


_BR = 2000


def _mm1_body(x_ref, w_ref, u_ref):
    u_ref[...] = jnp.dot(x_ref[...], w_ref[...],
                         preferred_element_type=jnp.float32)


def _pre_body(dp_ref, u_ref, dis_ref, hp_ref):
    deg = dp_ref[0] + dp_ref[1] + 1.0
    dis = lax.rsqrt(deg)
    dis_ref[...] = dis
    hp_ref[...] = u_ref[...] * dis


def _mid_body(s_ref, hp_ref, dis_ref, b_ref, w_ref, o_ref):
    dis = dis_ref[...]
    agg = s_ref[0] + s_ref[1] - hp_ref[...]
    t = jnp.maximum(dis * agg + b_ref[...], 0.0)
    u = jnp.dot(t, w_ref[...], preferred_element_type=jnp.float32)
    o_ref[...] = u * dis


def _post_body(s_ref, hp_ref, dis_ref, b_ref, o_ref):
    agg = s_ref[0] + s_ref[1] - hp_ref[...]
    o_ref[...] = dis_ref[...] * agg + b_ref[...]


def _mm1_call(x, W1):
    return pl.pallas_call(
        _mm1_body,
        grid=(N // _BR,),
        in_specs=[
            pl.BlockSpec((_BR, D), lambda i: (i, 0)),
            pl.BlockSpec((D, D), lambda i: (0, 0)),
        ],
        out_specs=pl.BlockSpec((_BR, D), lambda i: (i, 0)),
        out_shape=jax.ShapeDtypeStruct((N, D), jnp.float32),
    )(x, W1)


def _pre_call(degp, u1):
    return pl.pallas_call(
        _pre_body,
        grid=(N // _BR,),
        in_specs=[
            pl.BlockSpec((NC, _BR, 1), lambda i: (0, i, 0)),
            pl.BlockSpec((_BR, D), lambda i: (i, 0)),
        ],
        out_specs=[
            pl.BlockSpec((_BR, 1), lambda i: (i, 0)),
            pl.BlockSpec((_BR, D), lambda i: (i, 0)),
        ],
        out_shape=[
            jax.ShapeDtypeStruct((N, 1), jnp.float32),
            jax.ShapeDtypeStruct((N, D), jnp.float32),
        ],
    )(degp, u1)


def _mid_call(s1, hp1, dis, b1, W2):
    return pl.pallas_call(
        _mid_body,
        grid=(N // _BR,),
        in_specs=[
            pl.BlockSpec((NC, _BR, D), lambda i: (0, i, 0)),
            pl.BlockSpec((_BR, D), lambda i: (i, 0)),
            pl.BlockSpec((_BR, 1), lambda i: (i, 0)),
            pl.BlockSpec((1, D), lambda i: (0, 0)),
            pl.BlockSpec((D, D), lambda i: (0, 0)),
        ],
        out_specs=pl.BlockSpec((_BR, D), lambda i: (i, 0)),
        out_shape=jax.ShapeDtypeStruct((N, D), jnp.float32),
    )(s1, hp1, dis, b1, W2)


def _post_call(s2, hp2, dis, b2):
    return pl.pallas_call(
        _post_body,
        grid=(N // _BR,),
        in_specs=[
            pl.BlockSpec((NC, _BR, D), lambda i: (0, i, 0)),
            pl.BlockSpec((_BR, D), lambda i: (i, 0)),
            pl.BlockSpec((_BR, 1), lambda i: (i, 0)),
            pl.BlockSpec((1, D), lambda i: (0, 0)),
        ],
        out_specs=pl.BlockSpec((_BR, D), lambda i: (i, 0)),
        out_shape=jax.ShapeDtypeStruct((N, D), jnp.float32),
    )(s2, hp2, dis, b2)


def kernel(x, edge_index, W1, b1, W2, b2):
    src = edge_index[0].astype(jnp.int32)
    dst = edge_index[1].astype(jnp.int32)
    src4 = src.reshape(NC, NT, C_EDGE // CB, CB, K)
    dst4 = dst.reshape(NC, NT, C_EDGE // CB, CB, K)

    degp = _build_deg_kernel()(dst4)
    u1 = _mm1_call(x, W1)
    degp = degp[:, :N, None]
    dis, hp1 = _pre_call(degp, u1)
    s1 = _build_agg_kernel()(hp1, src4, dst4)
    hp2 = _mid_call(s1, hp1, dis, b1.reshape(1, D), W2)
    s2 = _build_agg_kernel()(hp2, src4, dst4)
    return _post_call(s2, hp2, dis, b2.reshape(1, D))


# speedup vs baseline: 1.0361x; 1.0255x over previous
import functools

import jax
import jax.numpy as jnp
from jax import lax
from jax.experimental import pallas as pl
from jax.experimental.pallas import tpu as pltpu
from jax.experimental.pallas import tpu_sc as plsc

N = 10000
E = 320000
D = 128
NC = 2
NT = 16
K = 80
C_EDGE = E // (NC * NT * K)
NB = 2
CB = 25
RPT = 640
RPT_LAST = N - (NT - 1) * RPT
DEG_PAD = 10240
DEG_TILE = DEG_PAD // NT


def _sc_mesh():
    return plsc.VectorSubcoreMesh(core_axis_name="c", subcore_axis_name="s")


@functools.cache
def _build_deg_kernel():
    return functools.partial(
        pl.kernel,
        out_type=jax.ShapeDtypeStruct((NC, DEG_PAD), jnp.float32),
        mesh=_sc_mesh(),
        scratch_types=[
            pltpu.VMEM((CB, K), jnp.int32),
            pltpu.VMEM((K,), jnp.float32),
            pltpu.VMEM((DEG_TILE,), jnp.float32),
            pltpu.VMEM_SHARED((DEG_PAD,), jnp.float32),
            pltpu.SemaphoreType.DMA,
        ],
    )(_deg_body)


def _deg_body(dst_hbm, out_hbm, idx_v, ones_v, zer_v, dacc, dsem):
    c = lax.axis_index("c")
    s = lax.axis_index("s")
    for i in range(K // 16):
        ones_v[pl.ds(i * 16, 16)] = jnp.ones((16,), jnp.float32)
    for i in range(DEG_TILE // 16):
        zer_v[pl.ds(i * 16, 16)] = jnp.zeros((16,), jnp.float32)
    d0 = pl.multiple_of(s * DEG_TILE, 8)
    pltpu.sync_copy(zer_v, dacc.at[pl.ds(d0, DEG_TILE)])
    plsc.subcore_barrier()

    def block(blk, carry):
        pltpu.sync_copy(dst_hbm.at[c, s, blk], idx_v)
        pltpu.async_copy(ones_v, dacc.at[idx_v.at[0]], dsem, add=True)

        def body(j, carry2):
            pltpu.async_copy(ones_v, dacc.at[idx_v.at[j]], dsem, add=True)
            pltpu.make_async_copy(ones_v, dacc.at[idx_v.at[j - 1]],
                                  dsem).wait()
            return carry2

        lax.fori_loop(1, CB, body, 0)
        pltpu.make_async_copy(ones_v, dacc.at[idx_v.at[CB - 1]], dsem).wait()
        return carry

    lax.fori_loop(0, C_EDGE // CB, block, 0)
    plsc.subcore_barrier()
    pltpu.sync_copy(dacc.at[pl.ds(d0, DEG_TILE)],
                    out_hbm.at[c, pl.ds(d0, DEG_TILE)])


@functools.cache
def _build_agg_kernel():
    return functools.partial(
        pl.kernel,
        out_type=jax.ShapeDtypeStruct((NC, N, D), jnp.float32),
        mesh=_sc_mesh(),
        scratch_types=[
            *[pltpu.VMEM((CB, K), jnp.int32) for _ in range(4)],
            *[pltpu.VMEM((K, D), jnp.float32) for _ in range(NB)],
            pltpu.VMEM_SHARED((N, D), jnp.float32),
            pltpu.SemaphoreType.DMA,
            pltpu.SemaphoreType.DMA,
        ],
    )(_agg_body)


def _agg_body(hp_hbm, src_hbm, dst_hbm, out_hbm, *rest):
    src_ib = rest[0:2]
    dst_ib = rest[2:4]
    bufs = rest[4:4 + NB]
    acc, gsem, isem = rest[4 + NB], rest[5 + NB], rest[6 + NB]
    c = lax.axis_index("c")
    s = lax.axis_index("s")
    r0 = pl.multiple_of(s * RPT, 8)

    @pl.when(s < NT - 1)
    def _stage_full():
        pltpu.sync_copy(hp_hbm.at[pl.ds(r0, RPT)], acc.at[pl.ds(r0, RPT)])

    @pl.when(s == NT - 1)
    def _stage_last():
        r = (NT - 1) * RPT
        pltpu.sync_copy(hp_hbm.at[pl.ds(r, RPT_LAST)],
                        acc.at[pl.ds(r, RPT_LAST)])

    pltpu.async_copy(src_hbm.at[c, s, 0], src_ib[0], isem)
    pltpu.async_copy(dst_hbm.at[c, s, 0], dst_ib[0], isem)
    pltpu.make_async_copy(src_hbm.at[c, s, 0], src_ib[0], isem).wait()
    pltpu.make_async_copy(dst_hbm.at[c, s, 0], dst_ib[0], isem).wait()
    plsc.subcore_barrier()

    n_blocks = C_EDGE // CB
    for blk in range(n_blocks):
        src_v = src_ib[blk % 2]
        dst_v = dst_ib[blk % 2]
        if blk + 1 < n_blocks:
            pltpu.async_copy(src_hbm.at[c, s, blk + 1],
                             src_ib[(blk + 1) % 2], isem)
            pltpu.async_copy(dst_hbm.at[c, s, blk + 1],
                             dst_ib[(blk + 1) % 2], isem)

        def _wait_gather(j, buf, src_v=src_v):
            pltpu.make_async_copy(hp_hbm.at[src_v.at[j]], buf, gsem).wait()

        pltpu.async_copy(hp_hbm.at[src_v.at[0]], bufs[0], gsem)

        def body(i, carry2, src_v=src_v, dst_v=dst_v,
                 _wait_gather=_wait_gather):
            j = 2 * i
            _wait_gather(j, bufs[0])
            pltpu.async_copy(hp_hbm.at[src_v.at[j + 1]], bufs[1], gsem)
            pltpu.sync_copy(bufs[0], acc.at[dst_v.at[j]], add=True)
            _wait_gather(j + 1, bufs[1])
            pltpu.async_copy(hp_hbm.at[src_v.at[j + 2]], bufs[0], gsem)
            pltpu.sync_copy(bufs[1], acc.at[dst_v.at[j + 1]], add=True)
            return carry2

        lax.fori_loop(0, (CB - 1) // 2, body, 0)
        _wait_gather(CB - 1, bufs[0])
        pltpu.sync_copy(bufs[0], acc.at[dst_v.at[CB - 1]], add=True)
        if blk + 1 < n_blocks:
            pltpu.make_async_copy(src_hbm.at[c, s, blk + 1],
                                  src_ib[(blk + 1) % 2], isem).wait()
            pltpu.make_async_copy(dst_hbm.at[c, s, blk + 1],
                                  dst_ib[(blk + 1) % 2], isem).wait()

    plsc.subcore_barrier()

    @pl.when(s < NT - 1)
    def _write_full():
        pltpu.sync_copy(acc.at[pl.ds(r0, RPT)], out_hbm.at[c, pl.ds(r0, RPT)])

    @pl.when(s == NT - 1)
    def _write_last():
        r = (NT - 1) * RPT
        pltpu.sync_copy(acc.at[pl.ds(r, RPT_LAST)],
                        out_hbm.at[c, pl.ds(r, RPT_LAST)])


_BR = 2000


def _mm1_body(x_ref, w_ref, u_ref):
    u_ref[...] = jnp.dot(x_ref[...], w_ref[...],
                         preferred_element_type=jnp.float32)


def _pre_body(dp_ref, u_ref, dis_ref, hp_ref):
    deg = dp_ref[0] + dp_ref[1] + 1.0
    dis = lax.rsqrt(deg)
    dis_ref[...] = dis
    hp_ref[...] = u_ref[...] * dis


def _mid_body(s_ref, hp_ref, dis_ref, b_ref, w_ref, o_ref):
    dis = dis_ref[...]
    agg = s_ref[0] + s_ref[1] - hp_ref[...]
    t = jnp.maximum(dis * agg + b_ref[...], 0.0)
    u = jnp.dot(t, w_ref[...], preferred_element_type=jnp.float32)
    o_ref[...] = u * dis


def _post_body(s_ref, hp_ref, dis_ref, b_ref, o_ref):
    agg = s_ref[0] + s_ref[1] - hp_ref[...]
    o_ref[...] = dis_ref[...] * agg + b_ref[...]


def _mm1_call(x, W1):
    return pl.pallas_call(
        _mm1_body,
        grid=(N // _BR,),
        in_specs=[
            pl.BlockSpec((_BR, D), lambda i: (i, 0)),
            pl.BlockSpec((D, D), lambda i: (0, 0)),
        ],
        out_specs=pl.BlockSpec((_BR, D), lambda i: (i, 0)),
        out_shape=jax.ShapeDtypeStruct((N, D), jnp.float32),
    )(x, W1)


def _pre_call(degp, u1):
    return pl.pallas_call(
        _pre_body,
        grid=(N // _BR,),
        in_specs=[
            pl.BlockSpec((NC, _BR, 1), lambda i: (0, i, 0)),
            pl.BlockSpec((_BR, D), lambda i: (i, 0)),
        ],
        out_specs=[
            pl.BlockSpec((_BR, 1), lambda i: (i, 0)),
            pl.BlockSpec((_BR, D), lambda i: (i, 0)),
        ],
        out_shape=[
            jax.ShapeDtypeStruct((N, 1), jnp.float32),
            jax.ShapeDtypeStruct((N, D), jnp.float32),
        ],
    )(degp, u1)


def _mid_call(s1, hp1, dis, b1, W2):
    return pl.pallas_call(
        _mid_body,
        grid=(N // _BR,),
        in_specs=[
            pl.BlockSpec((NC, _BR, D), lambda i: (0, i, 0)),
            pl.BlockSpec((_BR, D), lambda i: (i, 0)),
            pl.BlockSpec((_BR, 1), lambda i: (i, 0)),
            pl.BlockSpec((1, D), lambda i: (0, 0)),
            pl.BlockSpec((D, D), lambda i: (0, 0)),
        ],
        out_specs=pl.BlockSpec((_BR, D), lambda i: (i, 0)),
        out_shape=jax.ShapeDtypeStruct((N, D), jnp.float32),
    )(s1, hp1, dis, b1, W2)


def _post_call(s2, hp2, dis, b2):
    return pl.pallas_call(
        _post_body,
        grid=(N // _BR,),
        in_specs=[
            pl.BlockSpec((NC, _BR, D), lambda i: (0, i, 0)),
            pl.BlockSpec((_BR, D), lambda i: (i, 0)),
            pl.BlockSpec((_BR, 1), lambda i: (i, 0)),
            pl.BlockSpec((1, D), lambda i: (0, 0)),
        ],
        out_specs=pl.BlockSpec((_BR, D), lambda i: (i, 0)),
        out_shape=jax.ShapeDtypeStruct((N, D), jnp.float32),
    )(s2, hp2, dis, b2)


def kernel(x, edge_index, W1, b1, W2, b2):
    src = edge_index[0].astype(jnp.int32)
    dst = edge_index[1].astype(jnp.int32)
    src4 = src.reshape(NC, NT, C_EDGE // CB, CB, K)
    dst4 = dst.reshape(NC, NT, C_EDGE // CB, CB, K)

    degp = _build_deg_kernel()(dst4)
    u1 = _mm1_call(x, W1)
    degp = degp[:, :N, None]
    dis, hp1 = _pre_call(degp, u1)
    s1 = _build_agg_kernel()(hp1, src4, dst4)
    hp2 = _mid_call(s1, hp1, dis, b1.reshape(1, D), W2)
    s2 = _build_agg_kernel()(hp2, src4, dst4)
    return _post_call(s2, hp2, dis, b2.reshape(1, D))


# speedup vs baseline: 1.0392x; 1.0030x over previous
import functools

import jax
import jax.numpy as jnp
from jax import lax
from jax.experimental import pallas as pl
from jax.experimental.pallas import tpu as pltpu
from jax.experimental.pallas import tpu_sc as plsc

N = 10000
E = 320000
D = 128
NC = 2
NT = 16
K = 80
C_EDGE = E // (NC * NT * K)
NB = 2
CB = 25
RPT = 640
RPT_LAST = N - (NT - 1) * RPT
DEG_PAD = 10240
DEG_TILE = DEG_PAD // NT


def _sc_mesh():
    return plsc.VectorSubcoreMesh(core_axis_name="c", subcore_axis_name="s")


@functools.cache
def _build_deg_kernel():
    return functools.partial(
        pl.kernel,
        out_type=jax.ShapeDtypeStruct((NC, DEG_PAD), jnp.float32),
        mesh=_sc_mesh(),
        scratch_types=[
            pltpu.VMEM((CB, K), jnp.int32),
            pltpu.VMEM((K,), jnp.float32),
            pltpu.VMEM((DEG_TILE,), jnp.float32),
            pltpu.VMEM_SHARED((DEG_PAD,), jnp.float32),
            pltpu.SemaphoreType.DMA,
        ],
    )(_deg_body)


def _deg_body(dst_hbm, out_hbm, idx_v, ones_v, zer_v, dacc, dsem):
    c = lax.axis_index("c")
    s = lax.axis_index("s")
    for i in range(K // 16):
        ones_v[pl.ds(i * 16, 16)] = jnp.ones((16,), jnp.float32)
    for i in range(DEG_TILE // 16):
        zer_v[pl.ds(i * 16, 16)] = jnp.zeros((16,), jnp.float32)
    d0 = pl.multiple_of(s * DEG_TILE, 8)
    pltpu.sync_copy(zer_v, dacc.at[pl.ds(d0, DEG_TILE)])
    plsc.subcore_barrier()

    def block(blk, carry):
        pltpu.sync_copy(dst_hbm.at[c, s, blk], idx_v)
        pltpu.async_copy(ones_v, dacc.at[idx_v.at[0]], dsem, add=True)

        def body(j, carry2):
            pltpu.async_copy(ones_v, dacc.at[idx_v.at[j]], dsem, add=True)
            pltpu.make_async_copy(ones_v, dacc.at[idx_v.at[j - 1]],
                                  dsem).wait()
            return carry2

        lax.fori_loop(1, CB, body, 0)
        pltpu.make_async_copy(ones_v, dacc.at[idx_v.at[CB - 1]], dsem).wait()
        return carry

    lax.fori_loop(0, C_EDGE // CB, block, 0)
    plsc.subcore_barrier()
    pltpu.sync_copy(dacc.at[pl.ds(d0, DEG_TILE)],
                    out_hbm.at[c, pl.ds(d0, DEG_TILE)])


@functools.cache
def _build_agg_kernel():
    return functools.partial(
        pl.kernel,
        out_type=jax.ShapeDtypeStruct((NC, N, D), jnp.float32),
        mesh=_sc_mesh(),
        scratch_types=[
            *[pltpu.VMEM((CB, K), jnp.int32) for _ in range(4)],
            *[pltpu.VMEM((K, D), jnp.float32) for _ in range(NB)],
            pltpu.VMEM_SHARED((N, D), jnp.float32),
            pltpu.SemaphoreType.DMA,
            pltpu.SemaphoreType.DMA,
        ],
    )(_agg_body)


def _agg_body(hp_hbm, src_hbm, dst_hbm, out_hbm, *rest):
    src_ib = rest[0:2]
    dst_ib = rest[2:4]
    bufs = rest[4:4 + NB]
    acc, gsem, isem = rest[4 + NB], rest[5 + NB], rest[6 + NB]
    c = lax.axis_index("c")
    s = lax.axis_index("s")
    r0 = pl.multiple_of(s * RPT, 8)

    @pl.when(s < NT - 1)
    def _stage_full():
        pltpu.sync_copy(hp_hbm.at[pl.ds(r0, RPT)], acc.at[pl.ds(r0, RPT)])

    @pl.when(s == NT - 1)
    def _stage_last():
        r = (NT - 1) * RPT
        pltpu.sync_copy(hp_hbm.at[pl.ds(r, RPT_LAST)],
                        acc.at[pl.ds(r, RPT_LAST)])

    pltpu.async_copy(src_hbm.at[c, s, 0], src_ib[0], isem)
    pltpu.async_copy(dst_hbm.at[c, s, 0], dst_ib[0], isem)
    pltpu.make_async_copy(src_hbm.at[c, s, 0], src_ib[0], isem).wait()
    pltpu.make_async_copy(dst_hbm.at[c, s, 0], dst_ib[0], isem).wait()
    plsc.subcore_barrier()

    n_blocks = C_EDGE // CB
    for blk in range(n_blocks):
        src_v = src_ib[blk % 2]
        dst_v = dst_ib[blk % 2]
        if blk + 1 < n_blocks:
            pltpu.async_copy(src_hbm.at[c, s, blk + 1],
                             src_ib[(blk + 1) % 2], isem)
            pltpu.async_copy(dst_hbm.at[c, s, blk + 1],
                             dst_ib[(blk + 1) % 2], isem)

        def _wait_gather(j, buf, src_v=src_v):
            pltpu.make_async_copy(hp_hbm.at[src_v.at[j]], buf, gsem).wait()

        pltpu.async_copy(hp_hbm.at[src_v.at[0]], bufs[0], gsem)

        def body(i, carry2, src_v=src_v, dst_v=dst_v,
                 _wait_gather=_wait_gather):
            j = 2 * i
            _wait_gather(j, bufs[0])
            pltpu.async_copy(hp_hbm.at[src_v.at[j + 1]], bufs[1], gsem)
            pltpu.sync_copy(bufs[0], acc.at[dst_v.at[j]], add=True)
            _wait_gather(j + 1, bufs[1])
            pltpu.async_copy(hp_hbm.at[src_v.at[j + 2]], bufs[0], gsem)
            pltpu.sync_copy(bufs[1], acc.at[dst_v.at[j + 1]], add=True)
            return carry2

        lax.fori_loop(0, (CB - 1) // 2, body, 0)
        _wait_gather(CB - 1, bufs[0])
        pltpu.sync_copy(bufs[0], acc.at[dst_v.at[CB - 1]], add=True)
        if blk + 1 < n_blocks:
            pltpu.make_async_copy(src_hbm.at[c, s, blk + 1],
                                  src_ib[(blk + 1) % 2], isem).wait()
            pltpu.make_async_copy(dst_hbm.at[c, s, blk + 1],
                                  dst_ib[(blk + 1) % 2], isem).wait()

    plsc.subcore_barrier()

    @pl.when(s < NT - 1)
    def _write_full():
        pltpu.sync_copy(acc.at[pl.ds(r0, RPT)], out_hbm.at[c, pl.ds(r0, RPT)])

    @pl.when(s == NT - 1)
    def _write_last():
        r = (NT - 1) * RPT
        pltpu.sync_copy(acc.at[pl.ds(r, RPT_LAST)],
                        out_hbm.at[c, pl.ds(r, RPT_LAST)])


_BR = 2000


def _pre_body(dp_ref, x_ref, w_ref, dis_ref, hp_ref):
    deg = dp_ref[0] + dp_ref[1] + 1.0
    dis = lax.rsqrt(deg)
    dis_ref[...] = dis
    u = jnp.dot(x_ref[...], w_ref[...], preferred_element_type=jnp.float32)
    hp_ref[...] = u * dis


def _mid_body(s_ref, hp_ref, dis_ref, b_ref, w_ref, o_ref):
    dis = dis_ref[...]
    agg = s_ref[0] + s_ref[1] - hp_ref[...]
    t = jnp.maximum(dis * agg + b_ref[...], 0.0)
    u = jnp.dot(t, w_ref[...], preferred_element_type=jnp.float32)
    o_ref[...] = u * dis


def _post_body(s_ref, hp_ref, dis_ref, b_ref, o_ref):
    agg = s_ref[0] + s_ref[1] - hp_ref[...]
    o_ref[...] = dis_ref[...] * agg + b_ref[...]


def _pre_call(degp, x, W1):
    return pl.pallas_call(
        _pre_body,
        grid=(N // _BR,),
        in_specs=[
            pl.BlockSpec((NC, _BR, 1), lambda i: (0, i, 0)),
            pl.BlockSpec((_BR, D), lambda i: (i, 0)),
            pl.BlockSpec((D, D), lambda i: (0, 0)),
        ],
        out_specs=[
            pl.BlockSpec((_BR, 1), lambda i: (i, 0)),
            pl.BlockSpec((_BR, D), lambda i: (i, 0)),
        ],
        out_shape=[
            jax.ShapeDtypeStruct((N, 1), jnp.float32),
            jax.ShapeDtypeStruct((N, D), jnp.float32),
        ],
    )(degp, x, W1)


def _mid_call(s1, hp1, dis, b1, W2):
    return pl.pallas_call(
        _mid_body,
        grid=(N // _BR,),
        in_specs=[
            pl.BlockSpec((NC, _BR, D), lambda i: (0, i, 0)),
            pl.BlockSpec((_BR, D), lambda i: (i, 0)),
            pl.BlockSpec((_BR, 1), lambda i: (i, 0)),
            pl.BlockSpec((1, D), lambda i: (0, 0)),
            pl.BlockSpec((D, D), lambda i: (0, 0)),
        ],
        out_specs=pl.BlockSpec((_BR, D), lambda i: (i, 0)),
        out_shape=jax.ShapeDtypeStruct((N, D), jnp.float32),
    )(s1, hp1, dis, b1, W2)


def _post_call(s2, hp2, dis, b2):
    return pl.pallas_call(
        _post_body,
        grid=(N // _BR,),
        in_specs=[
            pl.BlockSpec((NC, _BR, D), lambda i: (0, i, 0)),
            pl.BlockSpec((_BR, D), lambda i: (i, 0)),
            pl.BlockSpec((_BR, 1), lambda i: (i, 0)),
            pl.BlockSpec((1, D), lambda i: (0, 0)),
        ],
        out_specs=pl.BlockSpec((_BR, D), lambda i: (i, 0)),
        out_shape=jax.ShapeDtypeStruct((N, D), jnp.float32),
    )(s2, hp2, dis, b2)


def kernel(x, edge_index, W1, b1, W2, b2):
    src = edge_index[0].astype(jnp.int32)
    dst = edge_index[1].astype(jnp.int32)
    src4 = src.reshape(NC, NT, C_EDGE // CB, CB, K)
    dst4 = dst.reshape(NC, NT, C_EDGE // CB, CB, K)

    degp = _build_deg_kernel()(dst4)
    degp = degp[:, :N, None]
    dis, hp1 = _pre_call(degp, x, W1)
    s1 = _build_agg_kernel()(hp1, src4, dst4)
    hp2 = _mid_call(s1, hp1, dis, b1.reshape(1, D), W2)
    s2 = _build_agg_kernel()(hp2, src4, dst4)
    return _post_call(s2, hp2, dis, b2.reshape(1, D))
